# baseline (speedup 1.0000x reference)
import jax
import jax.numpy as jnp
from jax import lax
from jax.experimental import pallas as pl
from jax.experimental.pallas import tpu as pltpu

_TK = 2048
_L = 128


def _vq_kernel(x_ref, p_ref, cb_ref, out_ref):
    rows = x_ref.shape[0]
    k_total = cb_ref.shape[0]

    h = jnp.dot(x_ref[...], p_ref[...], preferred_element_type=jnp.float32)
    n = jnp.sqrt(jnp.sum(h * h, axis=1, keepdims=True))
    hn = h / jnp.maximum(n, 1e-12)

    aug = jnp.concatenate([hn, jnp.ones_like(hn)], axis=1)

    mval = jnp.full((rows, _L), jnp.inf, dtype=jnp.float32)
    mcol = jnp.zeros((rows, _L), dtype=jnp.int32)
    for t in range(k_total // _TK):
        cbm = cb_ref[t * _TK:(t + 1) * _TK, :]
        rhs = jnp.concatenate([cbm * (-2.0), cbm * cbm], axis=1)
        d = lax.dot_general(aug, rhs, (((1,), (1,)), ((), ())),
                            preferred_element_type=jnp.float32,
                            precision=lax.Precision.HIGHEST)
        for c in range(_TK // _L):
            dc = d[:, c * _L:(c + 1) * _L]
            upd = dc < mval
            mval = jnp.where(upd, dc, mval)
            mcol = jnp.where(upd, t * (_TK // _L) + c, mcol)
    gmin = jnp.min(mval, axis=1, keepdims=True)
    kfull = mcol * _L + lax.broadcasted_iota(jnp.int32, (rows, _L), 1)
    ksel = jnp.where(mval == gmin, kfull, k_total)
    out_ref[...] = jnp.min(ksel, axis=1, keepdims=True)


def kernel(hidden_states, P, CB):
    B, T, D = hidden_states.shape
    NB, K, CD = CB.shape
    x = hidden_states.reshape(B * T, D)
    codes = pl.pallas_call(
        _vq_kernel,
        out_shape=jax.ShapeDtypeStruct((B * T, 1), jnp.int32),
    )(x, P[0], CB[0])
    return codes.reshape(B, NB, T)
